# baseline (speedup 1.0000x reference)
import functools

import jax
import jax.numpy as jnp
from jax import lax
from jax.experimental import pallas as pl
from jax.experimental.pallas import tpu as pltpu
from jax.experimental.pallas import tpu_sc as plsc

N_FRAG = 1600000
N_CELLS = 10000
N_GENES = 100

NC = 2
NS = 16
NW = NC * NS
PER_W = N_FRAG // NW
BATCH = 512
FULL_BATCHES = PER_W // BATCH
TAIL = PER_W - FULL_BATCHES * BATCH

HIST = 1024000
SLICE = HIST // NS
ZCHUNK = 16000

CHUNK = 128000


def _sc_hist_body(idx_hbm, parts0_hbm, parts1_hbm, idx_v, zbuf_v,
                  ones_v, hist_sh, ld_sem, sc_sem, z_sem):
    cid = lax.axis_index("c")
    sid = lax.axis_index("s")
    w = cid * NS + sid

    idx_i = idx_v.at[0]

    idx_dma = pltpu.async_copy(idx_hbm.at[pl.ds(w * PER_W, PER_W)], idx_i, ld_sem)

    def fill_ones(i, c):
        ones_v[pl.ds(i * 16, 16)] = jnp.ones((16,), jnp.float32)
        return c

    lax.fori_loop(0, BATCH // 16, fill_ones, 0)

    def fill_zero(i, c):
        zbuf_v[pl.ds(i * 16, 16)] = jnp.zeros((16,), jnp.float32)
        return c

    lax.fori_loop(0, ZCHUNK // 16, fill_zero, 0)

    def zero_hist(i, c):
        pltpu.async_copy(zbuf_v,
                         hist_sh.at[pl.ds(sid * SLICE + i * ZCHUNK, ZCHUNK)],
                         z_sem)
        return c

    lax.fori_loop(0, SLICE // ZCHUNK, zero_hist, 0)

    def zero_drain(i, c):
        pltpu.make_async_copy(
            zbuf_v, hist_sh.at[pl.ds(sid * SLICE, ZCHUNK)], z_sem).wait()
        return c

    lax.fori_loop(0, SLICE // ZCHUNK, zero_drain, 0)

    idx_dma.wait()
    plsc.subcore_barrier()

    def scatter(j, c):
        pltpu.async_copy(ones_v, hist_sh.at[idx_i.at[pl.ds(j * BATCH, BATCH)]],
                         sc_sem, add=True)
        return c

    lax.fori_loop(0, FULL_BATCHES, scatter, 0)
    pltpu.async_copy(ones_v.at[pl.ds(0, TAIL)],
                     hist_sh.at[idx_i.at[pl.ds(FULL_BATCHES * BATCH, TAIL)]],
                     sc_sem, add=True)

    def drain(j, c):
        pltpu.make_async_copy(
            ones_v, hist_sh.at[idx_i.at[pl.ds(0, BATCH)]], sc_sem).wait()
        return c

    lax.fori_loop(0, FULL_BATCHES, drain, 0)
    pltpu.make_async_copy(
        ones_v.at[pl.ds(0, TAIL)],
        hist_sh.at[idx_i.at[pl.ds(0, TAIL)]], sc_sem).wait()

    plsc.subcore_barrier()

    bufs = [zbuf_v, idx_v.bitcast(jnp.float32).at[0].at[pl.ds(0, ZCHUNK)]]
    n_chunks = SLICE // ZCHUNK

    def hop1(i, buf):
        return pltpu.async_copy(
            hist_sh.at[pl.ds(sid * SLICE + i * ZCHUNK, ZCHUNK)], buf, ld_sem)

    for pid, parts_hbm in enumerate([parts0_hbm, parts1_hbm]):

        @pl.when(cid == pid)
        def _(parts_hbm=parts_hbm):
            hop1(0, bufs[0]).wait()
            for i in range(n_chunks):
                d2 = pltpu.async_copy(
                    bufs[i % 2],
                    parts_hbm.at[pl.ds(sid * SLICE + i * ZCHUNK, ZCHUNK)],
                    sc_sem)
                if i + 1 < n_chunks:
                    hop1(i + 1, bufs[(i + 1) % 2]).wait()
                d2.wait()


_sc_hist = functools.partial(
    pl.kernel,
    out_type=[jax.ShapeDtypeStruct((HIST,), jnp.float32),
              jax.ShapeDtypeStruct((HIST,), jnp.float32)],
    mesh=plsc.VectorSubcoreMesh(core_axis_name="c", subcore_axis_name="s"),
    scratch_types=[
        pltpu.VMEM((1, PER_W), jnp.int32),
        pltpu.VMEM((ZCHUNK,), jnp.float32),
        pltpu.VMEM((BATCH,), jnp.float32),
        pltpu.VMEM_SHARED((HIST,), jnp.float32),
        pltpu.SemaphoreType.DMA,
        pltpu.SemaphoreType.DMA,
        pltpu.SemaphoreType.DMA,
    ],
)(_sc_hist_body)


def _tc_combine_body(p0_ref, p1_ref, scale_ref, bias_ref, out_ref):
    out_ref[...] = (p0_ref[...] + p1_ref[...]) * scale_ref[...] + bias_ref[...]


def kernel(coordinates, genemapping, local_cellxgene_ix, n_cells, n_genes,
           genes_oi, weight1, bias1):
    ix = local_cellxgene_ix.astype(jnp.int32)

    parts0, parts1 = _sc_hist(ix)

    scale_flat = jnp.tile(weight1[genes_oi, 0] * 10.0, CHUNK // N_GENES)
    bias_flat = jnp.tile(bias1[genes_oi], CHUNK // N_GENES)

    out_flat = pl.pallas_call(
        _tc_combine_body,
        grid=((N_CELLS * N_GENES + CHUNK - 1) // CHUNK,),
        in_specs=[
            pl.BlockSpec((CHUNK,), lambda i: (i,)),
            pl.BlockSpec((CHUNK,), lambda i: (i,)),
            pl.BlockSpec((CHUNK,), lambda i: (0,)),
            pl.BlockSpec((CHUNK,), lambda i: (0,)),
        ],
        out_specs=pl.BlockSpec((CHUNK,), lambda i: (i,)),
        out_shape=jax.ShapeDtypeStruct((N_CELLS * N_GENES,), jnp.float32),
    )(parts0, parts1, scale_flat, bias_flat)
    return out_flat.reshape(N_CELLS, N_GENES)


# speedup vs baseline: 1.0293x; 1.0293x over previous
import functools

import jax
import jax.numpy as jnp
from jax import lax
from jax.experimental import pallas as pl
from jax.experimental.pallas import tpu as pltpu
from jax.experimental.pallas import tpu_sc as plsc

N_FRAG = 1600000
N_CELLS = 10000
N_GENES = 100

NC = 2
NS = 16
NW = NC * NS
PER_W = N_FRAG // NW
BATCH = 256
FULL_BATCHES = PER_W // BATCH
TAIL = PER_W - FULL_BATCHES * BATCH

HIST = 1024000
SLICE = HIST // NS
ZCHUNK = 16000

CHUNK = 256000


def _sc_hist_body(idx_hbm, parts0_hbm, parts1_hbm, idx_v, zbuf_v,
                  ones_v, hist_sh, ld_sem, sc_sem, z_sem):
    cid = lax.axis_index("c")
    sid = lax.axis_index("s")
    w = cid * NS + sid

    idx_i = idx_v.at[0]

    idx_dma = pltpu.async_copy(idx_hbm.at[pl.ds(w * PER_W, PER_W)], idx_i, ld_sem)

    def fill_ones(i, c):
        ones_v[pl.ds(i * 16, 16)] = jnp.ones((16,), jnp.float32)
        return c

    lax.fori_loop(0, BATCH // 16, fill_ones, 0)

    def fill_zero(i, c):
        zbuf_v[pl.ds(i * 16, 16)] = jnp.zeros((16,), jnp.float32)
        return c

    lax.fori_loop(0, ZCHUNK // 16, fill_zero, 0)

    def zero_hist(i, c):
        pltpu.async_copy(zbuf_v,
                         hist_sh.at[pl.ds(sid * SLICE + i * ZCHUNK, ZCHUNK)],
                         z_sem)
        return c

    lax.fori_loop(0, SLICE // ZCHUNK, zero_hist, 0)

    def zero_drain(i, c):
        pltpu.make_async_copy(
            zbuf_v, hist_sh.at[pl.ds(sid * SLICE, ZCHUNK)], z_sem).wait()
        return c

    lax.fori_loop(0, SLICE // ZCHUNK, zero_drain, 0)

    idx_dma.wait()
    plsc.subcore_barrier()

    def scatter(j, c):
        pltpu.async_copy(ones_v, hist_sh.at[idx_i.at[pl.ds(j * BATCH, BATCH)]],
                         sc_sem, add=True)
        return c

    lax.fori_loop(0, FULL_BATCHES, scatter, 0)
    pltpu.async_copy(ones_v.at[pl.ds(0, TAIL)],
                     hist_sh.at[idx_i.at[pl.ds(FULL_BATCHES * BATCH, TAIL)]],
                     sc_sem, add=True)

    def drain(j, c):
        pltpu.make_async_copy(
            ones_v, hist_sh.at[idx_i.at[pl.ds(0, BATCH)]], sc_sem).wait()
        return c

    lax.fori_loop(0, FULL_BATCHES, drain, 0)
    pltpu.make_async_copy(
        ones_v.at[pl.ds(0, TAIL)],
        hist_sh.at[idx_i.at[pl.ds(0, TAIL)]], sc_sem).wait()

    plsc.subcore_barrier()

    bufs = [zbuf_v, idx_v.bitcast(jnp.float32).at[0].at[pl.ds(0, ZCHUNK)]]
    n_chunks = SLICE // ZCHUNK

    def hop1(i, buf):
        return pltpu.async_copy(
            hist_sh.at[pl.ds(sid * SLICE + i * ZCHUNK, ZCHUNK)], buf, ld_sem)

    for pid, parts_hbm in enumerate([parts0_hbm, parts1_hbm]):

        @pl.when(cid == pid)
        def _(parts_hbm=parts_hbm):
            hop1(0, bufs[0]).wait()
            for i in range(n_chunks):
                d2 = pltpu.async_copy(
                    bufs[i % 2],
                    parts_hbm.at[pl.ds(sid * SLICE + i * ZCHUNK, ZCHUNK)],
                    sc_sem)
                if i + 1 < n_chunks:
                    hop1(i + 1, bufs[(i + 1) % 2]).wait()
                d2.wait()


_sc_hist = functools.partial(
    pl.kernel,
    out_type=[jax.ShapeDtypeStruct((HIST,), jnp.float32),
              jax.ShapeDtypeStruct((HIST,), jnp.float32)],
    mesh=plsc.VectorSubcoreMesh(core_axis_name="c", subcore_axis_name="s"),
    scratch_types=[
        pltpu.VMEM((1, PER_W), jnp.int32),
        pltpu.VMEM((ZCHUNK,), jnp.float32),
        pltpu.VMEM((BATCH,), jnp.float32),
        pltpu.VMEM_SHARED((HIST,), jnp.float32),
        pltpu.SemaphoreType.DMA,
        pltpu.SemaphoreType.DMA,
        pltpu.SemaphoreType.DMA,
    ],
)(_sc_hist_body)


def _tc_combine_body(p0_ref, p1_ref, scale_ref, bias_ref, out_ref):
    out_ref[...] = (p0_ref[...] + p1_ref[...]) * scale_ref[...] + bias_ref[...]


def kernel(coordinates, genemapping, local_cellxgene_ix, n_cells, n_genes,
           genes_oi, weight1, bias1):
    ix = local_cellxgene_ix.astype(jnp.int32)

    parts0, parts1 = _sc_hist(ix)

    scale_flat = jnp.tile(weight1[genes_oi, 0] * 10.0, CHUNK // N_GENES)
    bias_flat = jnp.tile(bias1[genes_oi], CHUNK // N_GENES)

    out_flat = pl.pallas_call(
        _tc_combine_body,
        grid=((N_CELLS * N_GENES + CHUNK - 1) // CHUNK,),
        in_specs=[
            pl.BlockSpec((CHUNK,), lambda i: (i,)),
            pl.BlockSpec((CHUNK,), lambda i: (i,)),
            pl.BlockSpec((CHUNK,), lambda i: (0,)),
            pl.BlockSpec((CHUNK,), lambda i: (0,)),
        ],
        out_specs=pl.BlockSpec((CHUNK,), lambda i: (i,)),
        out_shape=jax.ShapeDtypeStruct((N_CELLS * N_GENES,), jnp.float32),
    )(parts0, parts1, scale_flat, bias_flat)
    return out_flat.reshape(N_CELLS, N_GENES)


# speedup vs baseline: 1.0412x; 1.0116x over previous
import functools

import jax
import jax.numpy as jnp
from jax import lax
from jax.experimental import pallas as pl
from jax.experimental.pallas import tpu as pltpu
from jax.experimental.pallas import tpu_sc as plsc

N_FRAG = 1600000
N_CELLS = 10000
N_GENES = 100

NC = 2
NS = 16
NW = NC * NS
PER_W = N_FRAG // NW
BATCH = 256
FULL_BATCHES = PER_W // BATCH
TAIL = PER_W - FULL_BATCHES * BATCH

HIST = 1024000
SLICE = HIST // NS
ZCHUNK = 16000

CHUNK = 512000


def _sc_hist_body(idx_hbm, parts0_hbm, parts1_hbm, idx_v, zbuf_v,
                  ones_v, hist_sh, ld_sem, sc_sem, z_sem):
    cid = lax.axis_index("c")
    sid = lax.axis_index("s")
    w = cid * NS + sid

    idx_i = idx_v.at[0]

    idx_dma = pltpu.async_copy(idx_hbm.at[pl.ds(w * PER_W, PER_W)], idx_i, ld_sem)

    def fill_ones(i, c):
        ones_v[pl.ds(i * 16, 16)] = jnp.ones((16,), jnp.float32)
        return c

    lax.fori_loop(0, BATCH // 16, fill_ones, 0)

    def fill_zero(i, c):
        zbuf_v[pl.ds(i * 16, 16)] = jnp.zeros((16,), jnp.float32)
        return c

    lax.fori_loop(0, ZCHUNK // 16, fill_zero, 0)

    def zero_hist(i, c):
        pltpu.async_copy(zbuf_v,
                         hist_sh.at[pl.ds(sid * SLICE + i * ZCHUNK, ZCHUNK)],
                         z_sem)
        return c

    lax.fori_loop(0, SLICE // ZCHUNK, zero_hist, 0)

    def zero_drain(i, c):
        pltpu.make_async_copy(
            zbuf_v, hist_sh.at[pl.ds(sid * SLICE, ZCHUNK)], z_sem).wait()
        return c

    lax.fori_loop(0, SLICE // ZCHUNK, zero_drain, 0)

    idx_dma.wait()
    plsc.subcore_barrier()

    def scatter(j, c):
        pltpu.async_copy(ones_v, hist_sh.at[idx_i.at[pl.ds(j * BATCH, BATCH)]],
                         sc_sem, add=True)
        return c

    lax.fori_loop(0, FULL_BATCHES, scatter, 0)
    pltpu.async_copy(ones_v.at[pl.ds(0, TAIL)],
                     hist_sh.at[idx_i.at[pl.ds(FULL_BATCHES * BATCH, TAIL)]],
                     sc_sem, add=True)

    def drain(j, c):
        pltpu.make_async_copy(
            ones_v, hist_sh.at[idx_i.at[pl.ds(0, BATCH)]], sc_sem).wait()
        return c

    lax.fori_loop(0, FULL_BATCHES, drain, 0)
    pltpu.make_async_copy(
        ones_v.at[pl.ds(0, TAIL)],
        hist_sh.at[idx_i.at[pl.ds(0, TAIL)]], sc_sem).wait()

    plsc.subcore_barrier()

    bufs = [zbuf_v, idx_v.bitcast(jnp.float32).at[0].at[pl.ds(0, ZCHUNK)]]
    n_chunks = SLICE // ZCHUNK

    def hop1(i, buf):
        return pltpu.async_copy(
            hist_sh.at[pl.ds(sid * SLICE + i * ZCHUNK, ZCHUNK)], buf, ld_sem)

    for pid, parts_hbm in enumerate([parts0_hbm, parts1_hbm]):

        @pl.when(cid == pid)
        def _(parts_hbm=parts_hbm):
            hop1(0, bufs[0]).wait()
            for i in range(n_chunks):
                d2 = pltpu.async_copy(
                    bufs[i % 2],
                    parts_hbm.at[pl.ds(sid * SLICE + i * ZCHUNK, ZCHUNK)],
                    sc_sem)
                if i + 1 < n_chunks:
                    hop1(i + 1, bufs[(i + 1) % 2]).wait()
                d2.wait()


_sc_hist = functools.partial(
    pl.kernel,
    out_type=[jax.ShapeDtypeStruct((HIST,), jnp.float32),
              jax.ShapeDtypeStruct((HIST,), jnp.float32)],
    mesh=plsc.VectorSubcoreMesh(core_axis_name="c", subcore_axis_name="s"),
    scratch_types=[
        pltpu.VMEM((1, PER_W), jnp.int32),
        pltpu.VMEM((ZCHUNK,), jnp.float32),
        pltpu.VMEM((BATCH,), jnp.float32),
        pltpu.VMEM_SHARED((HIST,), jnp.float32),
        pltpu.SemaphoreType.DMA,
        pltpu.SemaphoreType.DMA,
        pltpu.SemaphoreType.DMA,
    ],
)(_sc_hist_body)


def _tc_combine_body(p0_ref, p1_ref, scale_ref, bias_ref, out_ref):
    out_ref[...] = (p0_ref[...] + p1_ref[...]) * scale_ref[...] + bias_ref[...]


def kernel(coordinates, genemapping, local_cellxgene_ix, n_cells, n_genes,
           genes_oi, weight1, bias1):
    ix = local_cellxgene_ix.astype(jnp.int32)

    parts0, parts1 = _sc_hist(ix)

    scale_flat = jnp.tile(weight1[genes_oi, 0] * 10.0, CHUNK // N_GENES)
    bias_flat = jnp.tile(bias1[genes_oi], CHUNK // N_GENES)

    out_flat = pl.pallas_call(
        _tc_combine_body,
        grid=((N_CELLS * N_GENES + CHUNK - 1) // CHUNK,),
        in_specs=[
            pl.BlockSpec((CHUNK,), lambda i: (i,)),
            pl.BlockSpec((CHUNK,), lambda i: (i,)),
            pl.BlockSpec((CHUNK,), lambda i: (0,)),
            pl.BlockSpec((CHUNK,), lambda i: (0,)),
        ],
        out_specs=pl.BlockSpec((CHUNK,), lambda i: (i,)),
        out_shape=jax.ShapeDtypeStruct((N_CELLS * N_GENES,), jnp.float32),
    )(parts0, parts1, scale_flat, bias_flat)
    return out_flat.reshape(N_CELLS, N_GENES)
